# baseline (speedup 1.0000x reference)
import functools

import jax
import jax.numpy as jnp
from jax import lax
from jax.experimental import pallas as pl
from jax.experimental.pallas import tpu as pltpu
from jax.experimental.pallas import tpu_sc as plsc


def kernel(x, embed_table):
    b, n, m = x.shape
    e = embed_table.shape[-1]
    info = plsc.get_sparse_core_info()
    nw = info.num_cores * info.num_subcores
    per_w = b // nw
    rep = 8
    n_tab_dma = per_w // rep

    mesh = plsc.VectorSubcoreMesh(core_axis_name="c", subcore_axis_name="s")

    @functools.partial(
        pl.kernel,
        out_type=jax.ShapeDtypeStruct((b, n, m + e), jnp.float32),
        mesh=mesh,
        scratch_types=[
            pltpu.VMEM((rep, n, e), jnp.float32),
            pltpu.SemaphoreType.DMA,
            pltpu.SemaphoreType.DMA,
        ],
        compiler_params=pltpu.CompilerParams(use_tc_tiling_on_sc=False),
    )
    def run(x_hbm, tab_hbm, out_hbm, tab_v, sem_x, sem_t):
        wid = lax.axis_index("s") * info.num_cores + lax.axis_index("c")
        base = wid * per_w

        for r in range(rep):
            pltpu.sync_copy(tab_hbm, tab_v.at[r])

        cx = pltpu.make_async_copy(
            x_hbm.at[pl.ds(base, per_w)],
            out_hbm.at[pl.ds(base, per_w), :, pl.ds(0, m)],
            sem_x,
        )
        cx.start()

        def fire(i, carry):
            pltpu.make_async_copy(
                tab_v,
                out_hbm.at[pl.ds(base + i * rep, rep), :, pl.ds(m, e)],
                sem_t,
            ).start()
            return carry

        lax.fori_loop(0, n_tab_dma, fire, 0)

        def drain(i, carry):
            pltpu.make_async_copy(
                tab_v,
                out_hbm.at[pl.ds(base + i * rep, rep), :, pl.ds(m, e)],
                sem_t,
            ).wait()
            return carry

        lax.fori_loop(0, n_tab_dma, drain, 0)
        cx.wait()

    return run(x, embed_table)


# speedup vs baseline: 8.2713x; 8.2713x over previous
import functools

import jax
import jax.numpy as jnp
from jax import lax
from jax.experimental import pallas as pl
from jax.experimental.pallas import tpu as pltpu
from jax.experimental.pallas import tpu_sc as plsc

_NBUF = 4


def kernel(x, embed_table):
    b, n, m = x.shape
    e = embed_table.shape[-1]
    info = plsc.get_sparse_core_info()
    nw = info.num_cores * info.num_subcores
    per_w = b // nw
    ngroups = per_w // _NBUF

    mesh = plsc.VectorSubcoreMesh(core_axis_name="c", subcore_axis_name="s")

    @functools.partial(
        pl.kernel,
        out_type=jax.ShapeDtypeStruct((b, n, m + e), jnp.float32),
        mesh=mesh,
        scratch_types=[
            *[pltpu.VMEM((n, m + e), jnp.float32) for _ in range(_NBUF)],
            *[pltpu.SemaphoreType.DMA for _ in range(2 * _NBUF)],
        ],
        compiler_params=pltpu.CompilerParams(use_tc_tiling_on_sc=False),
    )
    def run(x_hbm, tab_hbm, out_hbm, *rest):
        bufs = rest[:_NBUF]
        sin = rest[_NBUF:2 * _NBUF]
        sout = rest[2 * _NBUF:]
        wid = lax.axis_index("s") * info.num_cores + lax.axis_index("c")
        base = wid * per_w

        for j in range(_NBUF):
            pltpu.sync_copy(tab_hbm, bufs[j].at[:, pl.ds(m, e)])

        def in_cp(j, bi):
            return pltpu.make_async_copy(
                x_hbm.at[bi], bufs[j].at[:, pl.ds(0, m)], sin[j])

        def out_cp(j, bi):
            return pltpu.make_async_copy(bufs[j], out_hbm.at[bi], sout[j])

        for j in range(_NBUF):
            in_cp(j, base + j).start()

        def outer(g, carry):
            gg = g * _NBUF
            for j in range(_NBUF):
                in_cp(j, base + gg + j).wait()
                out_cp(j, base + gg + j).start()
            for j in range(_NBUF):
                out_cp(j, base + gg + j).wait()
                nxt = gg + _NBUF + j

                @pl.when(nxt < per_w)
                def _():
                    in_cp(j, base + nxt).start()
            return carry

        lax.fori_loop(0, ngroups, outer, 0)

    return run(x, embed_table)
